# baseline (speedup 1.0000x reference)
import jax
import jax.numpy as jnp
from jax.experimental import pallas as pl
from jax.experimental.pallas import tpu as pltpu

_N = 1024
_BLK = 512
_KC = 128
_D_IN = 128
_D_OUT = 64


def _gat_body(adj_ref, x_ref, fc1_ref, fc2_ref, attn_ref, fc0_ref,
              out_ref, za_s, eq_s, eq01_s, es_s, es01_s):
    i = pl.program_id(0)
    default = jax.lax.Precision.DEFAULT
    highest = jax.lax.Precision.HIGHEST

    @pl.when(i == 0)
    def _init():
        z = jax.lax.dot_general(x_ref[...], fc1_ref[...],
                                (((1,), (1,)), ((), ())), precision=highest)
        za_s[:, 0:_D_OUT] = z
        col = jax.lax.broadcasted_iota(jnp.int32, (_N, _D_OUT), 1)
        za_s[:, _D_OUT:2 * _D_OUT] = jnp.where(col == 0, 1.0, 0.0)
        c = attn_ref[0, 2 * _D_OUT] * fc0_ref[0, 0]
        s_row = jax.lax.dot_general(attn_ref[:, 0:_D_OUT], z,
                                    (((1,), (1,)), ((), ())),
                                    precision=highest)
        q_col = c + jax.lax.dot_general(z, attn_ref[:, _D_OUT:2 * _D_OUT],
                                        (((1,), (1,)), ((), ())),
                                        precision=highest)
        es_s[...] = jnp.exp(s_row)
        es01_s[...] = jnp.exp(0.01 * s_row)
        eq_s[...] = jnp.exp(q_col)
        eq01_s[...] = jnp.exp(0.01 * q_col)

    eqb = eq_s[pl.ds(i * _BLK, _BLK), :]
    eq01b = eq01_s[pl.ds(i * _BLK, _BLK), :]
    agg = jnp.zeros((_BLK, 2 * _D_OUT), jnp.float32)
    for k in range(_N // _KC):
        ak = adj_ref[:, k * _KC:(k + 1) * _KC]
        t1 = eqb * es_s[:, k * _KC:(k + 1) * _KC]
        t2 = eq01b * es01_s[:, k * _KC:(k + 1) * _KC]
        p = jnp.where(ak > 0, jnp.maximum(t1, t2), 0.0)
        agg = agg + jax.lax.dot_general(
            p, za_s[k * _KC:(k + 1) * _KC, :], (((1,), (0,)), ((), ())),
            precision=default)
    zn = agg[:, 0:_D_OUT] / jnp.maximum(agg[:, _D_OUT:_D_OUT + 1], 1e-16)
    xb = x_ref[pl.ds(i * _BLK, _BLK), :]
    zi = jax.lax.dot_general(xb, fc2_ref[...], (((1,), (1,)), ((), ())),
                             precision=highest)
    out_ref[...] = jnp.maximum(zi + zn, 0.0)


def kernel(adjm, node_feats, fc0_w, fc1_w, fc2_w, attn_w, weights):
    del weights
    return pl.pallas_call(
        _gat_body,
        grid=(_N // _BLK,),
        in_specs=[
            pl.BlockSpec((_BLK, _N), lambda i: (i, 0)),
            pl.BlockSpec((_N, _D_IN), lambda i: (0, 0)),
            pl.BlockSpec((_D_OUT, _D_IN), lambda i: (0, 0)),
            pl.BlockSpec((_D_OUT, _D_IN), lambda i: (0, 0)),
            pl.BlockSpec((1, 2 * _D_OUT + 1), lambda i: (0, 0)),
            pl.BlockSpec((1, 1), lambda i: (0, 0)),
        ],
        out_specs=pl.BlockSpec((_BLK, _D_OUT), lambda i: (i, 0)),
        out_shape=jax.ShapeDtypeStruct((_N, _D_OUT), jnp.float32),
        scratch_shapes=[
            pltpu.VMEM((_N, 2 * _D_OUT), jnp.float32),
            pltpu.VMEM((_N, 1), jnp.float32),
            pltpu.VMEM((_N, 1), jnp.float32),
            pltpu.VMEM((1, _N), jnp.float32),
            pltpu.VMEM((1, _N), jnp.float32),
        ],
    )(adjm, node_feats, fc1_w, fc2_w, attn_w, fc0_w)


# speedup vs baseline: 1.0792x; 1.0792x over previous
import jax
import jax.numpy as jnp
from jax.experimental import pallas as pl
from jax.experimental.pallas import tpu as pltpu

_N = 1024
_BLK = 512
_KC = 512
_D_IN = 128
_D_OUT = 64


def _gat_body(adj_ref, x_ref, fc1_ref, fc2_ref, attn_ref, fc0_ref,
              out_ref, wcat_s, za_s, zi_s, g_s, es_s, es01_s):
    i = pl.program_id(0)
    default = jax.lax.Precision.DEFAULT
    highest = jax.lax.Precision.HIGHEST

    @pl.when(i == 0)
    def _init():
        c = attn_ref[0, 2 * _D_OUT] * fc0_ref[0, 0]
        v_s = jax.lax.dot_general(attn_ref[:, 0:_D_OUT], fc1_ref[...],
                                  (((1,), (0,)), ((), ())), precision=highest)
        v_q = jax.lax.dot_general(attn_ref[:, _D_OUT:2 * _D_OUT], fc1_ref[...],
                                  (((1,), (0,)), ((), ())), precision=highest)
        wcat_s[0:_D_OUT, :] = fc1_ref[...]
        wcat_s[_D_OUT:2 * _D_OUT, :] = fc2_ref[...]
        wcat_s[2 * _D_OUT:2 * _D_OUT + 1, :] = v_s
        wcat_s[2 * _D_OUT + 1:2 * _D_OUT + 2, :] = 0.99 * v_q
        wcat_s[2 * _D_OUT + 2:, :] = jnp.zeros((6, _D_IN), jnp.float32)
        big = jax.lax.dot_general(x_ref[...], wcat_s[...],
                                  (((1,), (1,)), ((), ())), precision=highest)
        za_s[:, 0:_D_OUT] = big[:, 0:_D_OUT]
        col = jax.lax.broadcasted_iota(jnp.int32, (_N, _D_OUT), 1)
        za_s[:, _D_OUT:2 * _D_OUT] = jnp.where(col == 0, 1.0, 0.0)
        zi_s[...] = big[:, _D_OUT:2 * _D_OUT]
        g_s[...] = jnp.exp(big[:, 2 * _D_OUT + 1:2 * _D_OUT + 2] + 0.99 * c)
        s_row = jax.lax.dot_general(v_s, x_ref[...], (((1,), (1,)), ((), ())),
                                    precision=highest)
        es_s[...] = jnp.exp(s_row)
        es01_s[...] = jnp.exp(0.01 * s_row)

    gb = g_s[pl.ds(i * _BLK, _BLK), :]
    agg = jnp.zeros((_BLK, 2 * _D_OUT), jnp.float32)
    for k in range(_N // _KC):
        ak = adj_ref[:, k * _KC:(k + 1) * _KC]
        t = gb * es_s[:, k * _KC:(k + 1) * _KC]
        m = jnp.maximum(t, es01_s[:, k * _KC:(k + 1) * _KC])
        p = jnp.where(ak > 0, m, 0.0)
        agg = agg + jax.lax.dot_general(
            p, za_s[k * _KC:(k + 1) * _KC, :], (((1,), (0,)), ((), ())),
            precision=default)
    zn = agg[:, 0:_D_OUT] / jnp.maximum(agg[:, _D_OUT:_D_OUT + 1], 1e-16)
    out_ref[...] = jnp.maximum(zi_s[pl.ds(i * _BLK, _BLK), :] + zn, 0.0)


def kernel(adjm, node_feats, fc0_w, fc1_w, fc2_w, attn_w, weights):
    del weights
    return pl.pallas_call(
        _gat_body,
        grid=(_N // _BLK,),
        in_specs=[
            pl.BlockSpec((_BLK, _N), lambda i: (i, 0)),
            pl.BlockSpec((_N, _D_IN), lambda i: (0, 0)),
            pl.BlockSpec((_D_OUT, _D_IN), lambda i: (0, 0)),
            pl.BlockSpec((_D_OUT, _D_IN), lambda i: (0, 0)),
            pl.BlockSpec((1, 2 * _D_OUT + 1), lambda i: (0, 0)),
            pl.BlockSpec((1, 1), lambda i: (0, 0)),
        ],
        out_specs=pl.BlockSpec((_BLK, _D_OUT), lambda i: (i, 0)),
        out_shape=jax.ShapeDtypeStruct((_N, _D_OUT), jnp.float32),
        scratch_shapes=[
            pltpu.VMEM((2 * _D_OUT + 8, _D_IN), jnp.float32),
            pltpu.VMEM((_N, 2 * _D_OUT), jnp.float32),
            pltpu.VMEM((_N, _D_OUT), jnp.float32),
            pltpu.VMEM((_N, 1), jnp.float32),
            pltpu.VMEM((1, _N), jnp.float32),
            pltpu.VMEM((1, _N), jnp.float32),
        ],
    )(adjm, node_feats, fc1_w, fc2_w, attn_w, fc0_w)


# speedup vs baseline: 1.2329x; 1.1425x over previous
import jax
import jax.numpy as jnp
from jax.experimental import pallas as pl
from jax.experimental.pallas import tpu as pltpu

_N = 1024
_BLK = 512
_KC = 512
_D_IN = 128
_D_OUT = 64


def _gat_body(adj_ref, x_ref, fc1_ref, fc2_ref, attn_ref, fc0_ref,
              out_ref, wcat_s, za_s, zi_s, g_s, es_s, es01_s):
    i = pl.program_id(0)
    default = jax.lax.Precision.DEFAULT

    @pl.when(i == 0)
    def _init():
        c = attn_ref[0, 2 * _D_OUT] * fc0_ref[0, 0]
        v_s = jax.lax.dot_general(attn_ref[:, 0:_D_OUT], fc1_ref[...],
                                  (((1,), (0,)), ((), ())), precision=default)
        v_q = jax.lax.dot_general(attn_ref[:, _D_OUT:2 * _D_OUT], fc1_ref[...],
                                  (((1,), (0,)), ((), ())), precision=default)
        wcat_s[0:_D_OUT, :] = fc1_ref[...]
        wcat_s[_D_OUT:2 * _D_OUT, :] = fc2_ref[...]
        wcat_s[2 * _D_OUT:2 * _D_OUT + 1, :] = v_s
        wcat_s[2 * _D_OUT + 1:2 * _D_OUT + 2, :] = 0.99 * v_q
        wcat_s[2 * _D_OUT + 2:, :] = jnp.zeros((6, _D_IN), jnp.float32)
        big = jax.lax.dot_general(x_ref[...], wcat_s[...],
                                  (((1,), (1,)), ((), ())), precision=default)
        za_s[:, 0:_D_OUT] = big[:, 0:_D_OUT]
        col = jax.lax.broadcasted_iota(jnp.int32, (_N, _D_OUT), 1)
        za_s[:, _D_OUT:2 * _D_OUT] = jnp.where(col == 0, 1.0, 0.0)
        zi_s[...] = big[:, _D_OUT:2 * _D_OUT]
        g_s[...] = jnp.exp(big[:, 2 * _D_OUT + 1:2 * _D_OUT + 2] + 0.99 * c)
        s_row = jax.lax.dot_general(v_s, x_ref[...], (((1,), (1,)), ((), ())),
                                    precision=default)
        es_s[...] = jnp.exp(s_row)
        es01_s[...] = jnp.exp(0.01 * s_row)

    gb = g_s[pl.ds(i * _BLK, _BLK), :]
    agg = jnp.zeros((_BLK, 2 * _D_OUT), jnp.float32)
    for k in range(_N // _KC):
        ak = adj_ref[:, k * _KC:(k + 1) * _KC]
        t = gb * es_s[:, k * _KC:(k + 1) * _KC]
        m = jnp.maximum(t, es01_s[:, k * _KC:(k + 1) * _KC])
        p = jnp.where(ak > 0, m, 0.0)
        agg = agg + jax.lax.dot_general(
            p, za_s[k * _KC:(k + 1) * _KC, :], (((1,), (0,)), ((), ())),
            precision=default)
    zn = agg[:, 0:_D_OUT] / jnp.maximum(agg[:, _D_OUT:_D_OUT + 1], 1e-16)
    out_ref[...] = jnp.maximum(zi_s[pl.ds(i * _BLK, _BLK), :] + zn, 0.0)


def kernel(adjm, node_feats, fc0_w, fc1_w, fc2_w, attn_w, weights):
    del weights
    return pl.pallas_call(
        _gat_body,
        grid=(_N // _BLK,),
        in_specs=[
            pl.BlockSpec((_BLK, _N), lambda i: (i, 0)),
            pl.BlockSpec((_N, _D_IN), lambda i: (0, 0)),
            pl.BlockSpec((_D_OUT, _D_IN), lambda i: (0, 0)),
            pl.BlockSpec((_D_OUT, _D_IN), lambda i: (0, 0)),
            pl.BlockSpec((1, 2 * _D_OUT + 1), lambda i: (0, 0)),
            pl.BlockSpec((1, 1), lambda i: (0, 0)),
        ],
        out_specs=pl.BlockSpec((_BLK, _D_OUT), lambda i: (i, 0)),
        out_shape=jax.ShapeDtypeStruct((_N, _D_OUT), jnp.float32),
        scratch_shapes=[
            pltpu.VMEM((2 * _D_OUT + 8, _D_IN), jnp.float32),
            pltpu.VMEM((_N, 2 * _D_OUT), jnp.float32),
            pltpu.VMEM((_N, _D_OUT), jnp.float32),
            pltpu.VMEM((_N, 1), jnp.float32),
            pltpu.VMEM((1, _N), jnp.float32),
            pltpu.VMEM((1, _N), jnp.float32),
        ],
    )(adjm, node_feats, fc1_w, fc2_w, attn_w, fc0_w)
